# baseline (speedup 1.0000x reference)
import functools

import jax
import jax.numpy as jnp
from jax import lax
from jax.experimental import pallas as pl
from jax.experimental.pallas import tpu as pltpu
from jax.experimental.pallas import tpu_sc as plsc

N = 10000
E = 320000
C = 32
M = 9
J = 9
H1, H2 = 64, 64
D_EDGE = 128
RESCALE = 5.0
F = M * C
HALF = F // 2

EDGE_BLK = 1000
NBLK = E // EDGE_BLK

NC = 2
NS = 16
CH = 128
NCHUNKS = E // CH
NPAD = 10240
ROWS_PER_TILE = NPAD // NS


def _edge_tc_kernel(xe_ref, wig_ref, env_ref, w1_ref, b1_ref, w2_ref,
                    b2_ref, w3_ref, b3_ref, outa_ref, outb_ref):
    xe = xe_ref[...]
    h = jax.nn.silu(jnp.dot(xe, w1_ref[...],
                            preferred_element_type=jnp.float32) + b1_ref[...])
    h = jax.nn.silu(jnp.dot(h, w2_ref[...],
                            preferred_element_type=jnp.float32) + b2_ref[...])
    emb = jnp.dot(h, w3_ref[...], preferred_element_type=jnp.float32) + b3_ref[...]
    emb = emb * (env_ref[...] * (1.0 / RESCALE))
    emb3 = emb.reshape(EDGE_BLK, M, C)
    wig = wig_ref[...]
    out3 = jnp.einsum('bjm,bmc->bjc', wig, emb3,
                      preferred_element_type=jnp.float32)
    outa_ref[...] = out3[:, :, :C // 2]
    outb_ref[...] = out3[:, :, C // 2:]


def _edge_messages(xe, wig3, env2, W1, b1, W2, b2, W3, b3):
    full = lambda s: pl.BlockSpec(s, lambda i: tuple(0 for _ in s))
    return pl.pallas_call(
        _edge_tc_kernel,
        grid=(NBLK,),
        in_specs=[
            pl.BlockSpec((EDGE_BLK, D_EDGE), lambda i: (i, 0)),
            pl.BlockSpec((EDGE_BLK, J, M), lambda i: (i, 0, 0)),
            pl.BlockSpec((EDGE_BLK, 1), lambda i: (i, 0)),
            full((D_EDGE, H1)), full((1, H1)),
            full((H1, H2)), full((1, H2)),
            full((H2, F)), full((1, F)),
        ],
        out_specs=[pl.BlockSpec((EDGE_BLK, J, C // 2), lambda i: (i, 0, 0)),
                   pl.BlockSpec((EDGE_BLK, J, C // 2), lambda i: (i, 0, 0))],
        out_shape=[jax.ShapeDtypeStruct((E, J, C // 2), jnp.float32),
                   jax.ShapeDtypeStruct((E, J, C // 2), jnp.float32)],
    )(xe, wig3, env2, W1, b1.reshape(1, H1), W2, b2.reshape(1, H2),
      W3, b3.reshape(1, F))


def _scatter_sc_body(msgs_a, msgs_b, dst, xa, xb, outa, outb,
                     idx_v, msg_v, acc_sh):
    pass


@functools.cache
def _make_scatter_sc():
    return pl.kernel(
        _scatter_sc_body,
        out_type=[jax.ShapeDtypeStruct((NPAD, HALF), jnp.float32),
                  jax.ShapeDtypeStruct((NPAD, HALF), jnp.float32)],
        mesh=plsc.VectorSubcoreMesh(core_axis_name="c", subcore_axis_name="s",
                                    num_cores=NC, num_subcores=NS),
        scratch_types=[
            pltpu.VMEM((CH,), jnp.int32),
            pltpu.VMEM((CH, HALF), jnp.float32),
            pltpu.VMEM_SHARED((NPAD, HALF), jnp.float32),
        ],
        compiler_params=pltpu.CompilerParams(use_tc_tiling_on_sc=False),
    )


def kernel(x, x_edge, edge_index, wigner_and_M_mapping_inv, edge_envelope,
           W1, b1, W2, b2, W3, b3):
    xp = jnp.pad(x, ((0, NPAD - N), (0, 0), (0, 0)))
    xa = xp[:, :, :C // 2].reshape(NPAD, HALF)
    xb = xp[:, :, C // 2:].reshape(NPAD, HALF)
    env2 = edge_envelope.reshape(E, 1)
    dst = edge_index[1]
    msgs_a, msgs_b = _edge_messages(x_edge, wigner_and_M_mapping_inv, env2,
                                    W1, b1, W2, b2, W3, b3)
    outa, outb = _make_scatter_sc()(msgs_a.reshape(E, HALF),
                                    msgs_b.reshape(E, HALF), dst, xa, xb)
    return jnp.concatenate([outa[:N].reshape(N, J, C // 2),
                            outb[:N].reshape(N, J, C // 2)], axis=2)


# speedup vs baseline: 2.3174x; 2.3174x over previous
import functools

import jax
import jax.numpy as jnp
from jax import lax
from jax.experimental import pallas as pl
from jax.experimental.pallas import tpu as pltpu
from jax.experimental.pallas import tpu_sc as plsc

N = 10000
E = 320000
C = 32
M = 9
J = 9
H1, H2 = 64, 64
D_EDGE = 128
RESCALE = 5.0
F = M * C
W_A = 128
W_B = 128
W_C = 32

EDGE_BLK = 1000
NBLK = E // EDGE_BLK

NC = 2
NS = 16
CH = 128
NCHUNKS = E // CH
NPAD = 10240
ROWS_PER_TILE = NPAD // NS


def _edge_tc_kernel(xe_ref, wig_ref, env_ref, w1_ref, b1_ref, w2_ref,
                    b2_ref, w3_ref, b3_ref, outa_ref, outb_ref, outc_ref):
    xe = xe_ref[...]
    h = jax.nn.silu(jnp.dot(xe, w1_ref[...],
                            preferred_element_type=jnp.float32) + b1_ref[...])
    h = jax.nn.silu(jnp.dot(h, w2_ref[...],
                            preferred_element_type=jnp.float32) + b2_ref[...])
    emb = jnp.dot(h, w3_ref[...], preferred_element_type=jnp.float32) + b3_ref[...]
    emb = emb * (env_ref[...] * (1.0 / RESCALE))
    emb3 = emb.reshape(EDGE_BLK, M, C)
    out3 = jnp.einsum('bjm,bmc->bjc', wig_ref[...], emb3,
                      preferred_element_type=jnp.float32)
    out2 = out3.reshape(EDGE_BLK, F)
    outa_ref[...] = out2[:, :W_A]
    outb_ref[...] = out2[:, W_A:W_A + W_B]
    outc_ref[:, :W_C] = out2[:, W_A + W_B:]


def _edge_messages(xe, wig3, env2, W1, b1, W2, b2, W3, b3):
    full = lambda s: pl.BlockSpec(s, lambda i: tuple(0 for _ in s))
    return pl.pallas_call(
        _edge_tc_kernel,
        grid=(NBLK,),
        in_specs=[
            pl.BlockSpec((EDGE_BLK, D_EDGE), lambda i: (i, 0)),
            pl.BlockSpec((EDGE_BLK, J, M), lambda i: (i, 0, 0)),
            pl.BlockSpec((EDGE_BLK, 1), lambda i: (i, 0)),
            full((D_EDGE, H1)), full((1, H1)),
            full((H1, H2)), full((1, H2)),
            full((H2, F)), full((1, F)),
        ],
        out_specs=[pl.BlockSpec((EDGE_BLK, 128), lambda i: (i, 0)),
                   pl.BlockSpec((EDGE_BLK, 128), lambda i: (i, 0)),
                   pl.BlockSpec((EDGE_BLK, 128), lambda i: (i, 0))],
        out_shape=[jax.ShapeDtypeStruct((E, 128), jnp.float32),
                   jax.ShapeDtypeStruct((E, 128), jnp.float32),
                   jax.ShapeDtypeStruct((E, 128), jnp.float32)],
    )(xe, wig3, env2, W1, b1.reshape(1, H1), W2, b2.reshape(1, H2),
      W3, b3.reshape(1, F))


def _scatter_sc_body(ma, mb, mc, dst, xa, xb, xc, outa, outb, outc,
                     idx_v, big_v, small_v, acc_big, acc_small):
    cid = lax.axis_index("c")
    wid = lax.axis_index("s")
    r0 = wid * ROWS_PER_TILE
    rows = pl.ds(r0, ROWS_PER_TILE)

    @pl.when(cid == 0)
    def _():
        pltpu.sync_copy(xa.at[rows], acc_big.at[rows])

    @pl.when(cid == 1)
    def _():
        pltpu.sync_copy(xb.at[rows], acc_big.at[rows])
        pltpu.sync_copy(xc.at[rows], acc_small.at[rows])

    plsc.subcore_barrier()

    nbase = NCHUNKS // NS
    rem = NCHUNKS % NS
    cnt = nbase + jnp.where(wid < rem, 1, 0)
    start = wid * nbase + jnp.minimum(wid, rem)

    def body(i, carry):
        e0 = (start + i) * CH
        pltpu.sync_copy(dst.at[pl.ds(e0, CH)], idx_v)

        @pl.when(cid == 0)
        def _():
            pltpu.sync_copy(ma.at[pl.ds(e0, CH)], big_v)
            pltpu.sync_copy(big_v, acc_big.at[idx_v], add=True)

        @pl.when(cid == 1)
        def _():
            pltpu.sync_copy(mb.at[pl.ds(e0, CH)], big_v)
            pltpu.sync_copy(mc.at[pl.ds(e0, CH), pl.ds(0, W_C)], small_v)
            pltpu.sync_copy(big_v, acc_big.at[idx_v], add=True)
            pltpu.sync_copy(small_v, acc_small.at[idx_v], add=True)

        return carry

    lax.fori_loop(0, cnt, body, 0)
    plsc.subcore_barrier()

    @pl.when(cid == 0)
    def _():
        pltpu.sync_copy(acc_big.at[rows], outa.at[rows])

    @pl.when(cid == 1)
    def _():
        pltpu.sync_copy(acc_big.at[rows], outb.at[rows])
        pltpu.sync_copy(acc_small.at[rows], outc.at[rows])


@functools.cache
def _make_scatter_sc():
    return pl.kernel(
        _scatter_sc_body,
        out_type=[jax.ShapeDtypeStruct((NPAD, 128), jnp.float32),
                  jax.ShapeDtypeStruct((NPAD, 128), jnp.float32),
                  jax.ShapeDtypeStruct((NPAD, W_C), jnp.float32)],
        mesh=plsc.VectorSubcoreMesh(core_axis_name="c", subcore_axis_name="s",
                                    num_cores=NC, num_subcores=NS),
        scratch_types=[
            pltpu.VMEM((CH,), jnp.int32),
            pltpu.VMEM((CH, 128), jnp.float32),
            pltpu.VMEM((CH, W_C), jnp.float32),
            pltpu.VMEM_SHARED((NPAD, 128), jnp.float32),
            pltpu.VMEM_SHARED((NPAD, W_C), jnp.float32),
        ],
        compiler_params=pltpu.CompilerParams(use_tc_tiling_on_sc=False),
    )


def kernel(x, x_edge, edge_index, wigner_and_M_mapping_inv, edge_envelope,
           W1, b1, W2, b2, W3, b3):
    x2 = jnp.pad(x.reshape(N, F), ((0, NPAD - N), (0, 0)))
    xa = x2[:, :W_A]
    xb = x2[:, W_A:W_A + W_B]
    xc = x2[:, W_A + W_B:]
    env2 = edge_envelope.reshape(E, 1)
    dst = edge_index[1]
    ma, mb, mc = _edge_messages(x_edge, wigner_and_M_mapping_inv, env2,
                                W1, b1, W2, b2, W3, b3)
    outa, outb, outc = _make_scatter_sc()(ma, mb, mc, dst, xa, xb, xc)
    return jnp.concatenate([outa[:N], outb[:N], outc[:N]],
                           axis=1).reshape(N, J, C)


# speedup vs baseline: 2.9959x; 1.2928x over previous
import functools

import jax
import jax.numpy as jnp
from jax import lax
from jax.experimental import pallas as pl
from jax.experimental.pallas import tpu as pltpu
from jax.experimental.pallas import tpu_sc as plsc

N = 10000
E = 320000
C = 32
M = 9
J = 9
H1, H2 = 64, 64
D_EDGE = 128
RESCALE = 5.0
F = M * C
W_A = 128
W_B = 128
W_C = 32

EDGE_BLK = 1280
NBLK = E // EDGE_BLK

NC = 2
NS = 16
CH = 128
NCHUNKS = E // CH
NPAD = 10240
ROWS_PER_TILE = NPAD // NS


def _edge_tc_kernel(xe_ref, wig_ref, env_ref, e3_ref, w1_ref, b1_ref, w2_ref,
                    b2_ref, w3_ref, b3_ref, outa_ref, outb_ref, outc_ref):
    xe = xe_ref[...]
    h = jax.nn.silu(jnp.dot(xe, w1_ref[...],
                            preferred_element_type=jnp.float32) + b1_ref[...])
    h = jax.nn.silu(jnp.dot(h, w2_ref[...],
                            preferred_element_type=jnp.float32) + b2_ref[...])
    emb = jnp.dot(h, w3_ref[...], preferred_element_type=jnp.float32) + b3_ref[...]
    emb3 = emb.reshape(EDGE_BLK, M, C)
    wig_s = wig_ref[...] * (env_ref[...] * (1.0 / RESCALE))
    wig3 = lax.dot_general(wig_s, e3_ref[...], (((0,), (0,)), ((), ())),
                           preferred_element_type=jnp.float32).reshape(
                               EDGE_BLK, J, M)
    out3 = jnp.einsum('bjm,bmc->bjc', wig3, emb3,
                      preferred_element_type=jnp.float32)
    out2 = out3.reshape(EDGE_BLK, F)
    outa_ref[...] = out2[:, :W_A]
    outb_ref[...] = out2[:, W_A:W_A + W_B]
    outc_ref[:, :W_C] = out2[:, W_A + W_B:]


def _edge_messages(xe, wig3, env2, W1, b1, W2, b2, W3, b3):
    full = lambda s: pl.BlockSpec(s, lambda i: tuple(0 for _ in s))
    return pl.pallas_call(
        _edge_tc_kernel,
        grid=(NBLK,),
        in_specs=[
            pl.BlockSpec((EDGE_BLK, D_EDGE), lambda i: (i, 0)),
            pl.BlockSpec((J * M, EDGE_BLK), lambda i: (0, i)),
            pl.BlockSpec((1, EDGE_BLK), lambda i: (0, i)),
            full((J * M, J * M)),
            full((D_EDGE, H1)), full((1, H1)),
            full((H1, H2)), full((1, H2)),
            full((H2, F)), full((1, F)),
        ],
        out_specs=[pl.BlockSpec((EDGE_BLK, 128), lambda i: (i, 0)),
                   pl.BlockSpec((EDGE_BLK, 128), lambda i: (i, 0)),
                   pl.BlockSpec((EDGE_BLK, 128), lambda i: (i, 0))],
        out_shape=[jax.ShapeDtypeStruct((E, 128), jnp.float32),
                   jax.ShapeDtypeStruct((E, 128), jnp.float32),
                   jax.ShapeDtypeStruct((E, 128), jnp.float32)],
    )(xe, wig3, env2, jnp.eye(J * M, dtype=jnp.float32),
      W1, b1.reshape(1, H1), W2, b2.reshape(1, H2),
      W3, b3.reshape(1, F))


def _scatter_sc_body(ma, mb, mc, dst, xa, xb, xc, outa, outb, outc,
                     idx_v, big_v, small_v, acc_big, acc_small):
    cid = lax.axis_index("c")
    wid = lax.axis_index("s")
    r0 = wid * ROWS_PER_TILE
    rows = pl.ds(r0, ROWS_PER_TILE)

    @pl.when(cid == 0)
    def _():
        pltpu.sync_copy(xa.at[rows], acc_big.at[rows])

    @pl.when(cid == 1)
    def _():
        pltpu.sync_copy(xb.at[rows], acc_big.at[rows])
        pltpu.sync_copy(xc.at[rows], acc_small.at[rows])

    plsc.subcore_barrier()

    nbase = NCHUNKS // NS
    rem = NCHUNKS % NS
    cnt = nbase + jnp.where(wid < rem, 1, 0)
    start = wid * nbase + jnp.minimum(wid, rem)

    def body(i, carry):
        e0 = (start + i) * CH
        pltpu.sync_copy(dst.at[pl.ds(e0, CH)], idx_v)

        @pl.when(cid == 0)
        def _():
            pltpu.sync_copy(ma.at[pl.ds(e0, CH)], big_v)
            pltpu.sync_copy(big_v, acc_big.at[idx_v], add=True)

        @pl.when(cid == 1)
        def _():
            pltpu.sync_copy(mb.at[pl.ds(e0, CH)], big_v)
            pltpu.sync_copy(mc.at[pl.ds(e0, CH), pl.ds(0, W_C)], small_v)
            pltpu.sync_copy(big_v, acc_big.at[idx_v], add=True)
            pltpu.sync_copy(small_v, acc_small.at[idx_v], add=True)

        return carry

    lax.fori_loop(0, cnt, body, 0)
    plsc.subcore_barrier()

    @pl.when(cid == 0)
    def _():
        pltpu.sync_copy(acc_big.at[rows], outa.at[rows])

    @pl.when(cid == 1)
    def _():
        pltpu.sync_copy(acc_big.at[rows], outb.at[rows])
        pltpu.sync_copy(acc_small.at[rows], outc.at[rows])


@functools.cache
def _make_scatter_sc():
    return pl.kernel(
        _scatter_sc_body,
        out_type=[jax.ShapeDtypeStruct((NPAD, 128), jnp.float32),
                  jax.ShapeDtypeStruct((NPAD, 128), jnp.float32),
                  jax.ShapeDtypeStruct((NPAD, W_C), jnp.float32)],
        mesh=plsc.VectorSubcoreMesh(core_axis_name="c", subcore_axis_name="s",
                                    num_cores=NC, num_subcores=NS),
        scratch_types=[
            pltpu.VMEM((CH,), jnp.int32),
            pltpu.VMEM((CH, 128), jnp.float32),
            pltpu.VMEM((CH, W_C), jnp.float32),
            pltpu.VMEM_SHARED((NPAD, 128), jnp.float32),
            pltpu.VMEM_SHARED((NPAD, W_C), jnp.float32),
        ],
        compiler_params=pltpu.CompilerParams(use_tc_tiling_on_sc=False),
    )


def kernel(x, x_edge, edge_index, wigner_and_M_mapping_inv, edge_envelope,
           W1, b1, W2, b2, W3, b3):
    x2 = jnp.pad(x.reshape(N, F), ((0, NPAD - N), (0, 0)))
    xa = x2[:, :W_A]
    xb = x2[:, W_A:W_A + W_B]
    xc = x2[:, W_A + W_B:]
    wigT = wigner_and_M_mapping_inv.transpose(1, 2, 0).reshape(J * M, E)
    envr = edge_envelope.reshape(1, E)
    dst = edge_index[1]
    ma, mb, mc = _edge_messages(x_edge, wigT, envr,
                                W1, b1, W2, b2, W3, b3)
    outa, outb, outc = _make_scatter_sc()(ma, mb, mc, dst, xa, xb, xc)
    return jnp.concatenate([outa[:N], outb[:N], outc[:N]],
                           axis=1).reshape(N, J, C)


# speedup vs baseline: 3.2733x; 1.0926x over previous
import functools

import jax
import jax.numpy as jnp
from jax import lax
from jax.experimental import pallas as pl
from jax.experimental.pallas import tpu as pltpu
from jax.experimental.pallas import tpu_sc as plsc

N = 10000
E = 320000
C = 32
M = 9
J = 9
H1, H2 = 64, 64
D_EDGE = 128
RESCALE = 5.0
F = M * C
W_A = 128
W_B = 128
W_C = 32

EDGE_BLK = 1280
NBLK = E // EDGE_BLK

NC = 2
NS = 16
CH = 128
NCHUNKS = E // CH
NPAD = 10240
ROWS_PER_TILE = NPAD // NS


def _edge_tc_kernel(xe_ref, wig_ref, env_ref, e3_ref, w1_ref, b1_ref, w2_ref,
                    b2_ref, w3_ref, b3_ref, outa_ref, outb_ref, outc_ref):
    xe = xe_ref[...]
    h = jax.nn.silu(jnp.dot(xe, w1_ref[...],
                            preferred_element_type=jnp.float32) + b1_ref[...])
    h = jax.nn.silu(jnp.dot(h, w2_ref[...],
                            preferred_element_type=jnp.float32) + b2_ref[...])
    emb = jnp.dot(h, w3_ref[...], preferred_element_type=jnp.float32) + b3_ref[...]
    emb3 = emb.reshape(EDGE_BLK, M, C)
    wig_s = wig_ref[...] * (env_ref[...] * (1.0 / RESCALE))
    wig3 = lax.dot_general(wig_s.astype(jnp.bfloat16), e3_ref[...],
                           (((0,), (0,)), ((), ())),
                           preferred_element_type=jnp.float32).reshape(
                               EDGE_BLK, J, M)
    out3 = jnp.einsum('bjm,bmc->bjc', wig3.astype(jnp.bfloat16),
                      emb3.astype(jnp.bfloat16),
                      preferred_element_type=jnp.float32)
    out2 = out3.reshape(EDGE_BLK, F)
    outa_ref[...] = out2[:, :W_A]
    outb_ref[...] = out2[:, W_A:W_A + W_B]
    outc_ref[:, :W_C] = out2[:, W_A + W_B:]


def _edge_messages(xe, wig3, env2, W1, b1, W2, b2, W3, b3):
    full = lambda s: pl.BlockSpec(s, lambda i: tuple(0 for _ in s))
    return pl.pallas_call(
        _edge_tc_kernel,
        grid=(NBLK,),
        in_specs=[
            pl.BlockSpec((EDGE_BLK, D_EDGE), lambda i: (i, 0)),
            pl.BlockSpec((J * M, EDGE_BLK), lambda i: (0, i)),
            pl.BlockSpec((1, EDGE_BLK), lambda i: (0, i)),
            full((J * M, J * M)),
            full((D_EDGE, H1)), full((1, H1)),
            full((H1, H2)), full((1, H2)),
            full((H2, F)), full((1, F)),
        ],
        out_specs=[pl.BlockSpec((EDGE_BLK, 128), lambda i: (i, 0)),
                   pl.BlockSpec((EDGE_BLK, 128), lambda i: (i, 0)),
                   pl.BlockSpec((EDGE_BLK, 128), lambda i: (i, 0))],
        out_shape=[jax.ShapeDtypeStruct((E, 128), jnp.float32),
                   jax.ShapeDtypeStruct((E, 128), jnp.float32),
                   jax.ShapeDtypeStruct((E, 128), jnp.float32)],
    )(xe, wig3, env2, jnp.eye(J * M, dtype=jnp.bfloat16),
      W1, b1.reshape(1, H1), W2, b2.reshape(1, H2),
      W3, b3.reshape(1, F))


def _scatter_sc_body(ma, mb, mc, dst, xa, xb, xc, outa, outb, outc,
                     idx_v, big_v, small_v, acc_big, acc_small):
    cid = lax.axis_index("c")
    wid = lax.axis_index("s")
    r0 = wid * ROWS_PER_TILE
    rows = pl.ds(r0, ROWS_PER_TILE)

    @pl.when(cid == 0)
    def _():
        pltpu.sync_copy(xa.at[rows], acc_big.at[rows])

    @pl.when(cid == 1)
    def _():
        pltpu.sync_copy(xb.at[rows], acc_big.at[rows])
        pltpu.sync_copy(xc.at[rows], acc_small.at[rows])

    plsc.subcore_barrier()

    nbase = NCHUNKS // NS
    rem = NCHUNKS % NS
    cnt = nbase + jnp.where(wid < rem, 1, 0)
    start = wid * nbase + jnp.minimum(wid, rem)

    def body(i, carry):
        e0 = (start + i) * CH
        pltpu.sync_copy(dst.at[pl.ds(e0, CH)], idx_v)

        @pl.when(cid == 0)
        def _():
            pltpu.sync_copy(ma.at[pl.ds(e0, CH)], big_v)
            pltpu.sync_copy(big_v, acc_big.at[idx_v], add=True)

        @pl.when(cid == 1)
        def _():
            pltpu.sync_copy(mb.at[pl.ds(e0, CH)], big_v)
            pltpu.sync_copy(mc.at[pl.ds(e0, CH), pl.ds(0, W_C)], small_v)
            pltpu.sync_copy(big_v, acc_big.at[idx_v], add=True)
            pltpu.sync_copy(small_v, acc_small.at[idx_v], add=True)

        return carry

    lax.fori_loop(0, cnt, body, 0)
    plsc.subcore_barrier()

    @pl.when(cid == 0)
    def _():
        pltpu.sync_copy(acc_big.at[rows], outa.at[rows])

    @pl.when(cid == 1)
    def _():
        pltpu.sync_copy(acc_big.at[rows], outb.at[rows])
        pltpu.sync_copy(acc_small.at[rows], outc.at[rows])


@functools.cache
def _make_scatter_sc():
    return pl.kernel(
        _scatter_sc_body,
        out_type=[jax.ShapeDtypeStruct((NPAD, 128), jnp.float32),
                  jax.ShapeDtypeStruct((NPAD, 128), jnp.float32),
                  jax.ShapeDtypeStruct((NPAD, W_C), jnp.float32)],
        mesh=plsc.VectorSubcoreMesh(core_axis_name="c", subcore_axis_name="s",
                                    num_cores=NC, num_subcores=NS),
        scratch_types=[
            pltpu.VMEM((CH,), jnp.int32),
            pltpu.VMEM((CH, 128), jnp.float32),
            pltpu.VMEM((CH, W_C), jnp.float32),
            pltpu.VMEM_SHARED((NPAD, 128), jnp.float32),
            pltpu.VMEM_SHARED((NPAD, W_C), jnp.float32),
        ],
        compiler_params=pltpu.CompilerParams(use_tc_tiling_on_sc=False),
    )


def kernel(x, x_edge, edge_index, wigner_and_M_mapping_inv, edge_envelope,
           W1, b1, W2, b2, W3, b3):
    x2 = jnp.pad(x.reshape(N, F), ((0, NPAD - N), (0, 0)))
    xa = x2[:, :W_A]
    xb = x2[:, W_A:W_A + W_B]
    xc = x2[:, W_A + W_B:]
    wigT = wigner_and_M_mapping_inv.transpose(1, 2, 0).reshape(J * M, E)
    envr = edge_envelope.reshape(1, E)
    dst = edge_index[1]
    ma, mb, mc = _edge_messages(x_edge, wigT, envr,
                                W1, b1, W2, b2, W3, b3)
    outa, outb, outc = _make_scatter_sc()(ma, mb, mc, dst, xa, xb, xc)
    return jnp.concatenate([outa[:N], outb[:N], outc[:N]],
                           axis=1).reshape(N, J, C)
